# Initial kernel scaffold; baseline (speedup 1.0000x reference)
import jax
import jax.numpy as jnp
from jax.experimental import pallas as pl


def kernel(emb, h, c, edge_index, type_n, W_iou, U_iou, b_iou, W_f, U_f_w, U_f_b, b_f):
    raise NotImplementedError("write your pallas kernel here")


# speedup vs baseline: 5.2115x; 5.2115x over previous
import functools

import jax
import jax.numpy as jnp
from jax import lax
from jax.experimental import pallas as pl
from jax.experimental.pallas import tpu as pltpu
from jax.experimental.pallas import tpu_sc as plsc

NC = 2
NS = 16
CH = 128
CW = 64


def _sc_typed_segment_sums(tbl_q0, tbl_q1, srcp2, sidxp, zeros, rows_total,
                           chunks):
    slab = rows_total // NS
    epw = chunks * CH
    mesh = plsc.VectorSubcoreMesh(core_axis_name="c", subcore_axis_name="s",
                                  num_cores=NC, num_subcores=NS)

    @functools.partial(
        pl.kernel,
        out_type=jax.ShapeDtypeStruct((NC, 2, rows_total, CW), jnp.float32),
        mesh=mesh,
        scratch_types=[
            pltpu.VMEM((epw,), jnp.int32),
            pltpu.VMEM((chunks, CH), jnp.int32),
            pltpu.VMEM((CH, CW), jnp.float32),
            pltpu.VMEM_SHARED((rows_total, CW), jnp.float32),
            pltpu.SemaphoreType.DMA,
        ],
        compiler_params=pltpu.CompilerParams(use_tc_tiling_on_sc=False),
    )
    def k(t0_hbm, t1_hbm, srcp2_hbm, sidxp_hbm, zeros_hbm, out_hbm,
          src_v, sidx_v, buf, acc, sem):
        cid = lax.axis_index("c")
        sid = lax.axis_index("s")
        pltpu.sync_copy(srcp2_hbm.at[cid, sid], src_v)
        pltpu.sync_copy(sidxp_hbm.at[sid], sidx_v)
        for q, tbl in ((0, t0_hbm), (1, t1_hbm)):
            pltpu.sync_copy(zeros_hbm, acc.at[pl.ds(sid * slab, slab)])
            plsc.subcore_barrier()

            @pl.loop(0, chunks)
            def _(j):
                idx = src_v.at[pl.ds(j * CH, CH)]
                pltpu.async_copy(tbl.at[idx], buf, sem).wait()
                pltpu.sync_copy(buf, acc.at[sidx_v.at[j]], add=True)

            plsc.subcore_barrier()
            pltpu.sync_copy(acc.at[pl.ds(sid * slab, slab)],
                            out_hbm.at[cid, q, pl.ds(sid * slab, slab)])

    return k(tbl_q0, tbl_q1, srcp2, sidxp, zeros)


def _tc_dense(emb, hiou, ct0, ct1, WiouT, UiouT, biou, WfT, UfwT, fbias):
    n, d = emb.shape
    h = d
    bn = 2000
    grid = (n // bn,)

    def body(emb_r, hiou_r, ct0_r, ct1_r, wiouT, uiouT, biou_r, wfT, ufwT,
             fb_r, out_r):
        embv = emb_r[...]
        hiouv = hiou_r[...]
        x = jnp.dot(embv, wfT[...], preferred_element_type=jnp.float32)
        f = jnp.dot(hiouv, ufwT[...], preferred_element_type=jnp.float32)
        f = f + fb_r[...]
        s0 = jax.nn.sigmoid(x + f[:, :h])
        s1 = jax.nn.sigmoid(x + f[:, h:])
        c_cell = s0 * ct0_r[...] + s1 * ct1_r[...]
        iou = (jnp.dot(embv, wiouT[...], preferred_element_type=jnp.float32)
               + jnp.dot(hiouv, uiouT[...], preferred_element_type=jnp.float32)
               + biou_r[...])
        i = jax.nn.sigmoid(iou[:, :h])
        o = jax.nn.sigmoid(iou[:, h:2 * h])
        u = jnp.tanh(iou[:, 2 * h:])
        c_new = i * u + c_cell
        h_new = o * jnp.tanh(c_new)
        out_r[...] = jnp.concatenate([h_new, c_new], axis=1)

    return pl.pallas_call(
        body,
        grid=grid,
        in_specs=[
            pl.BlockSpec((bn, d), lambda i: (i, 0)),
            pl.BlockSpec((bn, 2 * h), lambda i: (i, 0)),
            pl.BlockSpec((bn, h), lambda i: (i, 0)),
            pl.BlockSpec((bn, h), lambda i: (i, 0)),
            pl.BlockSpec((d, 3 * h), lambda i: (0, 0)),
            pl.BlockSpec((2 * h, 3 * h), lambda i: (0, 0)),
            pl.BlockSpec((1, 3 * h), lambda i: (0, 0)),
            pl.BlockSpec((d, h), lambda i: (0, 0)),
            pl.BlockSpec((2 * h, 2 * h), lambda i: (0, 0)),
            pl.BlockSpec((1, 2 * h), lambda i: (0, 0)),
        ],
        out_specs=pl.BlockSpec((bn, 2 * h), lambda i: (i, 0)),
        out_shape=jax.ShapeDtypeStruct((n, 2 * h), jnp.float32),
    )(emb, hiou, ct0, ct1, WiouT, UiouT, biou, WfT, UfwT, fbias)


def kernel(emb, h, c, edge_index, type_n, W_iou, U_iou, b_iou, W_f, U_f_w,
           U_f_b, b_f):
    n, d = emb.shape
    hh = h.shape[1]
    e = edge_index.shape[1]
    src = edge_index[0]
    dst = edge_index[1]

    sidx = dst + n * jnp.take(type_n, src)

    chunks = -(-e // (NS * CH))
    ep = NS * chunks * CH
    pad = ep - e
    rows_total = -(-(2 * n + 1) // (NS * 8)) * (NS * 8)
    pad_rows = rows_total - 2 * n
    if pad:
        ar = jnp.arange(pad, dtype=jnp.int32)
        src_p = jnp.concatenate([src, ar % jnp.int32(min(n, 4096))])
        sidx_p = jnp.concatenate([sidx, 2 * n + (ar % jnp.int32(max(pad_rows, 1)))])
    else:
        src_p, sidx_p = src, sidx
    srcp = src_p.reshape(NS, chunks * CH)
    srcp2 = jnp.stack([srcp, srcp + n])
    sidxp = sidx_p.reshape(NS, chunks, CH)

    tbl_q0 = jnp.concatenate([h[:, :CW], c[:, :CW]], axis=0)
    tbl_q1 = jnp.concatenate([h[:, CW:], c[:, CW:]], axis=0)
    zeros = jnp.zeros((rows_total // NS, CW), jnp.float32)

    sums = _sc_typed_segment_sums(tbl_q0, tbl_q1, srcp2, sidxp, zeros,
                                  rows_total, chunks)

    ht0 = jnp.concatenate([sums[0, 0, :n], sums[0, 1, :n]], axis=1)
    ht1 = jnp.concatenate([sums[0, 0, n:2 * n], sums[0, 1, n:2 * n]], axis=1)
    ct0 = jnp.concatenate([sums[1, 0, :n], sums[1, 1, :n]], axis=1)
    ct1 = jnp.concatenate([sums[1, 0, n:2 * n], sums[1, 1, n:2 * n]], axis=1)
    hiou = jnp.concatenate([ht0, ht1], axis=1)

    fbias = U_f_b.reshape(1, 2 * hh) + jnp.concatenate([b_f, b_f], axis=1)
    return _tc_dense(emb, hiou, ct0, ct1, W_iou.T, U_iou.T, b_iou,
                     W_f.T, U_f_w.T, fbias)
